# baseline (speedup 1.0000x reference)
import functools

import jax
import jax.numpy as jnp
from jax import lax
from jax.experimental import pallas as pl
from jax.experimental.pallas import tpu as pltpu
from jax.experimental.pallas import tpu_sc as plsc

N = 10000
D = 128
NP = 10112
ROWS_PER_TILE = NP // 16
E = 320000
CHUNK = 128
CHUNKS_PER_TILE = 81
EDGES_PER_TILE = CHUNKS_PER_TILE * CHUNK
EP = EDGES_PER_TILE * 32
ROW_BLOCK = 632
NSLOT = 3


def _linear_body(x_ref, w_ref, b_ref, c_ref, o_ref):
    acc = lax.dot_general(x_ref[...], w_ref[...], (((1,), (1,)), ((), ())),
                          preferred_element_type=jnp.float32)
    o_ref[...] = (acc + b_ref[...]) * c_ref[0]


def _linear(x, W, b, c):
    return pl.pallas_call(
        _linear_body,
        grid=(NP // ROW_BLOCK,),
        in_specs=[
            pl.BlockSpec((ROW_BLOCK, D), lambda i: (i, 0)),
            pl.BlockSpec((D, D), lambda i: (0, 0)),
            pl.BlockSpec((1, D), lambda i: (0, 0)),
            pl.BlockSpec(memory_space=pltpu.SMEM),
        ],
        out_specs=pl.BlockSpec((ROW_BLOCK, D), lambda i: (i, 0)),
        out_shape=jax.ShapeDtypeStruct((NP, D), jnp.float32),
    )(x, W, b, c)


def _mid_body(p_ref, w_ref, b_ref, c_ref, o_ref):
    h = jnp.maximum(p_ref[0] + p_ref[1], 0.0)
    acc = lax.dot_general(h, w_ref[...], (((1,), (1,)), ((), ())),
                          preferred_element_type=jnp.float32)
    o_ref[...] = (acc + b_ref[...]) * c_ref[0]


def _mid(p, W, b, c):
    return pl.pallas_call(
        _mid_body,
        grid=(NP // ROW_BLOCK,),
        in_specs=[
            pl.BlockSpec((2, ROW_BLOCK, D), lambda i: (0, i, 0)),
            pl.BlockSpec((D, D), lambda i: (0, 0)),
            pl.BlockSpec((1, D), lambda i: (0, 0)),
            pl.BlockSpec(memory_space=pltpu.SMEM),
        ],
        out_specs=pl.BlockSpec((ROW_BLOCK, D), lambda i: (i, 0)),
        out_shape=jax.ShapeDtypeStruct((NP, D), jnp.float32),
    )(p, W, b, c)


def _final_body(q_ref, o_ref):
    o = q_ref[0] + q_ref[1]
    m = jnp.max(o, axis=1, keepdims=True)
    e = jnp.exp(o - m)
    s = jnp.sum(e, axis=1, keepdims=True)
    o_ref[...] = o - m - jnp.log(s)


def _final(q):
    return pl.pallas_call(
        _final_body,
        grid=(NP // ROW_BLOCK,),
        in_specs=[pl.BlockSpec((2, ROW_BLOCK, D), lambda i: (0, i, 0))],
        out_specs=pl.BlockSpec((ROW_BLOCK, D), lambda i: (i, 0)),
        out_shape=jax.ShapeDtypeStruct((NP, D), jnp.float32),
    )(q)


_SC_MESH = plsc.VectorSubcoreMesh(core_axis_name="c", subcore_axis_name="s")


@functools.partial(
    pl.kernel,
    mesh=_SC_MESH,
    out_type=jax.ShapeDtypeStruct((2, NP, D), jnp.float32),
    scratch_types=[
        pltpu.VMEM((NSLOT, CHUNK), jnp.int32),
        pltpu.VMEM((NSLOT, CHUNK), jnp.int32),
        pltpu.VMEM((NSLOT, CHUNK, D), jnp.float32),
        pltpu.VMEM_SHARED((NP, D), jnp.float32),
        pltpu.SemaphoreType.DMA((NSLOT,)),
        pltpu.SemaphoreType.DMA((NSLOT,)),
        pltpu.SemaphoreType.DMA((NSLOT,)),
        pltpu.SemaphoreType.DMA((NSLOT,)),
    ],
)
def _sc_scatter(y_hbm, z_hbm, src_hbm, dst_hbm, out_hbm,
                sidx, didx, rows, accum, isems, isemd, gsem, ssem):
    c = lax.axis_index("c")
    s = lax.axis_index("s")
    wid = s * 2 + c
    r0 = s * ROWS_PER_TILE
    e0 = wid * EDGES_PER_TILE

    @pl.when(c == 0)
    def _():
        pltpu.sync_copy(y_hbm.at[pl.ds(r0, ROWS_PER_TILE)],
                        accum.at[pl.ds(r0, ROWS_PER_TILE)])

    @pl.when(c == 1)
    def _():
        pltpu.sync_copy(z_hbm.at[pl.ds(r0, ROWS_PER_TILE)],
                        accum.at[pl.ds(r0, ROWS_PER_TILE)])

    plsc.subcore_barrier()

    def fire_sidx(k, j):
        pltpu.async_copy(src_hbm.at[pl.ds(e0 + k * CHUNK, CHUNK)],
                         sidx.at[j], isems.at[j])

    def wait_sidx(k, j):
        pltpu.make_async_copy(src_hbm.at[pl.ds(e0 + k * CHUNK, CHUNK)],
                              sidx.at[j], isems.at[j]).wait()

    def fire_didx(k, j):
        pltpu.async_copy(dst_hbm.at[pl.ds(e0 + k * CHUNK, CHUNK)],
                         didx.at[j], isemd.at[j])

    def wait_didx(k, j):
        pltpu.make_async_copy(dst_hbm.at[pl.ds(e0 + k * CHUNK, CHUNK)],
                              didx.at[j], isemd.at[j]).wait()

    def fire_gather(j):
        pltpu.async_copy(y_hbm.at[sidx.at[j]], rows.at[j], gsem.at[j])

    def wait_gather(j):
        pltpu.make_async_copy(y_hbm.at[sidx.at[j]], rows.at[j],
                              gsem.at[j]).wait()

    def fire_scatter(j):
        pltpu.async_copy(rows.at[j], accum.at[didx.at[j]], ssem.at[j],
                         add=True)

    def wait_scatter(j):
        pltpu.make_async_copy(rows.at[j], accum.at[didx.at[j]],
                              ssem.at[j]).wait()

    pltpu.sync_copy(src_hbm.at[pl.ds(e0, CHUNK)], sidx.at[0])
    fire_sidx(1, 1)
    fire_didx(0, 0)
    fire_gather(0)

    def step(i, carry):
        for j in range(NSLOT):
            k = i * NSLOT + j
            j1 = (j + 1) % NSLOT
            j2 = (j + 2) % NSLOT

            @pl.when(k >= 2)
            def _():
                wait_scatter(j1)

            @pl.when(k + 1 < CHUNKS_PER_TILE)
            def _():
                wait_sidx(k + 1, j1)
                fire_gather(j1)
                fire_didx(k + 1, j1)

            @pl.when(k + 2 < CHUNKS_PER_TILE)
            def _():
                fire_sidx(k + 2, j2)

            wait_gather(j)
            wait_didx(k, j)
            fire_scatter(j)
        return carry

    lax.fori_loop(0, CHUNKS_PER_TILE // NSLOT, step, 0)

    wait_scatter((CHUNKS_PER_TILE - 2) % NSLOT)
    wait_scatter((CHUNKS_PER_TILE - 1) % NSLOT)

    plsc.subcore_barrier()
    pltpu.sync_copy(accum.at[pl.ds(r0, ROWS_PER_TILE)],
                    out_hbm.at[c].at[pl.ds(r0, ROWS_PER_TILE)])


def kernel(x, edge_index, W1, b1, Wa1, ba1, c1, W2, b2, Wa2, ba2, c2):
    src = edge_index[0]
    dst = edge_index[1]
    pad_e = EP - E
    src_p = jnp.concatenate([src, jnp.zeros((pad_e,), src.dtype)])
    dst_p = jnp.concatenate([dst, jnp.full((pad_e,), N, dst.dtype)])

    x_p = jnp.pad(x, ((0, NP - N), (0, 0)))
    z = jnp.zeros((NP, D), jnp.float32)
    c1s = -jnp.abs(c1)
    c2s = -jnp.abs(c2)

    y1 = _linear(x_p, W1, b1.reshape(1, D), c1s)
    p = _sc_scatter(y1, z, src_p, dst_p)
    y2 = _mid(p, W2, b2.reshape(1, D), c2s)
    q = _sc_scatter(y2, z, src_p, dst_p)
    o = _final(q)
    return o[:N]


# speedup vs baseline: 1.2531x; 1.2531x over previous
import functools

import jax
import jax.numpy as jnp
from jax import lax
from jax.experimental import pallas as pl
from jax.experimental.pallas import tpu as pltpu
from jax.experimental.pallas import tpu_sc as plsc

N = 10000
D = 128
NP = 10112
ROWS_PER_TILE = NP // 16
E = 320000
CHUNK = 128
CHUNKS_PER_TILE = 80
EDGES_PER_TILE = CHUNKS_PER_TILE * CHUNK
EP = EDGES_PER_TILE * 32
ROW_BLOCK = 632
HALF = CHUNKS_PER_TILE // 2


def _linear_body(x_ref, w_ref, b_ref, c_ref, o_ref):
    acc = lax.dot_general(x_ref[...], w_ref[...], (((1,), (1,)), ((), ())),
                          preferred_element_type=jnp.float32)
    o_ref[...] = (acc + b_ref[...]) * c_ref[0]


def _linear(x, W, b, c):
    return pl.pallas_call(
        _linear_body,
        grid=(NP // ROW_BLOCK,),
        in_specs=[
            pl.BlockSpec((ROW_BLOCK, D), lambda i: (i, 0)),
            pl.BlockSpec((D, D), lambda i: (0, 0)),
            pl.BlockSpec((1, D), lambda i: (0, 0)),
            pl.BlockSpec(memory_space=pltpu.SMEM),
        ],
        out_specs=pl.BlockSpec((ROW_BLOCK, D), lambda i: (i, 0)),
        out_shape=jax.ShapeDtypeStruct((NP, D), jnp.float32),
    )(x, W, b, c)


def _mid_body(p_ref, w_ref, b_ref, c_ref, o_ref):
    h = jnp.maximum(p_ref[0] + p_ref[1], 0.0)
    acc = lax.dot_general(h, w_ref[...], (((1,), (1,)), ((), ())),
                          preferred_element_type=jnp.float32)
    o_ref[...] = (acc + b_ref[...]) * c_ref[0]


def _mid(p, W, b, c):
    return pl.pallas_call(
        _mid_body,
        grid=(NP // ROW_BLOCK,),
        in_specs=[
            pl.BlockSpec((2, ROW_BLOCK, D), lambda i: (0, i, 0)),
            pl.BlockSpec((D, D), lambda i: (0, 0)),
            pl.BlockSpec((1, D), lambda i: (0, 0)),
            pl.BlockSpec(memory_space=pltpu.SMEM),
        ],
        out_specs=pl.BlockSpec((ROW_BLOCK, D), lambda i: (i, 0)),
        out_shape=jax.ShapeDtypeStruct((NP, D), jnp.float32),
    )(p, W, b, c)


def _final_body(q_ref, o_ref):
    o = q_ref[0] + q_ref[1]
    m = jnp.max(o, axis=1, keepdims=True)
    e = jnp.exp(o - m)
    s = jnp.sum(e, axis=1, keepdims=True)
    o_ref[...] = o - m - jnp.log(s)


def _final(q):
    return pl.pallas_call(
        _final_body,
        grid=(NP // ROW_BLOCK,),
        in_specs=[pl.BlockSpec((2, ROW_BLOCK, D), lambda i: (0, i, 0))],
        out_specs=pl.BlockSpec((ROW_BLOCK, D), lambda i: (i, 0)),
        out_shape=jax.ShapeDtypeStruct((NP, D), jnp.float32),
    )(q)


_SC_MESH = plsc.VectorSubcoreMesh(core_axis_name="c", subcore_axis_name="s")


@functools.partial(
    pl.kernel,
    mesh=_SC_MESH,
    out_type=jax.ShapeDtypeStruct((2, NP, D), jnp.float32),
    scratch_types=[
        pltpu.VMEM((HALF, CHUNK), jnp.int32),
        pltpu.VMEM((HALF, CHUNK), jnp.int32),
        pltpu.VMEM((2, CHUNK, D), jnp.float32),
        pltpu.VMEM_SHARED((NP, D), jnp.float32),
        pltpu.SemaphoreType.DMA((2,)),
        pltpu.SemaphoreType.DMA((2,)),
    ],
)
def _sc_scatter(y_hbm, z_hbm, src2d_hbm, dst2d_hbm, out_hbm,
                sidx, didx, rows, accum, gsem, ssem):
    c = lax.axis_index("c")
    s = lax.axis_index("s")
    wid = s * 2 + c
    r0 = s * ROWS_PER_TILE
    ch0 = wid * CHUNKS_PER_TILE

    @pl.when(c == 0)
    def _():
        pltpu.sync_copy(y_hbm.at[pl.ds(r0, ROWS_PER_TILE)],
                        accum.at[pl.ds(r0, ROWS_PER_TILE)])

    @pl.when(c == 1)
    def _():
        pltpu.sync_copy(z_hbm.at[pl.ds(r0, ROWS_PER_TILE)],
                        accum.at[pl.ds(r0, ROWS_PER_TILE)])

    plsc.subcore_barrier()

    def fire_gather(kk, j):
        pltpu.async_copy(y_hbm.at[sidx.at[kk]], rows.at[j], gsem.at[j])

    def wait_gather(kk, j):
        pltpu.make_async_copy(y_hbm.at[sidx.at[kk]], rows.at[j],
                              gsem.at[j]).wait()

    def fire_scatter(kk, j):
        pltpu.async_copy(rows.at[j], accum.at[didx.at[kk]], ssem.at[j],
                         add=True)

    def wait_scatter(kk, j):
        pltpu.make_async_copy(rows.at[j], accum.at[didx.at[kk]],
                              ssem.at[j]).wait()

    for phase in range(2):
        pltpu.sync_copy(
            src2d_hbm.at[pl.ds(ch0 + phase * HALF, HALF)], sidx)
        pltpu.sync_copy(
            dst2d_hbm.at[pl.ds(ch0 + phase * HALF, HALF)], didx)

        fire_gather(0, 0)
        fire_gather(1, 1)

        def pair(i, carry):
            kk = i * 2 + 1
            wait_scatter(kk - 1, 0)
            fire_gather(kk + 1, 0)
            wait_gather(kk, 1)
            fire_scatter(kk, 1)
            wait_scatter(kk, 1)
            fire_gather(kk + 2, 1)
            wait_gather(kk + 1, 0)
            fire_scatter(kk + 1, 0)
            return carry

        wait_gather(0, 0)
        fire_scatter(0, 0)
        lax.fori_loop(0, HALF // 2 - 1, pair, 0)

        kk = HALF - 1
        wait_gather(kk, 1)
        fire_scatter(kk, 1)
        wait_scatter(kk - 1, 0)
        wait_scatter(kk, 1)

    plsc.subcore_barrier()
    pltpu.sync_copy(accum.at[pl.ds(r0, ROWS_PER_TILE)],
                    out_hbm.at[c].at[pl.ds(r0, ROWS_PER_TILE)])


def kernel(x, edge_index, W1, b1, Wa1, ba1, c1, W2, b2, Wa2, ba2, c2):
    src = edge_index[0]
    dst = edge_index[1]
    pad_e = EP - E
    src_p = jnp.concatenate([src, jnp.zeros((pad_e,), src.dtype)])
    dst_p = jnp.concatenate([dst, jnp.full((pad_e,), N, dst.dtype)])
    src2d = src_p.reshape(CHUNKS_PER_TILE * 32, CHUNK)
    dst2d = dst_p.reshape(CHUNKS_PER_TILE * 32, CHUNK)

    x_p = jnp.pad(x, ((0, NP - N), (0, 0)))
    z = jnp.zeros((NP, D), jnp.float32)
    c1s = -jnp.abs(c1)
    c2s = -jnp.abs(c2)

    y1 = _linear(x_p, W1, b1.reshape(1, D), c1s)
    p = _sc_scatter(y1, z, src2d, dst2d)
    y2 = _mid(p, W2, b2.reshape(1, D), c2s)
    q = _sc_scatter(y2, z, src2d, dst2d)
    o = _final(q)
    return o[:N]


# speedup vs baseline: 1.2779x; 1.0198x over previous
import functools

import jax
import jax.numpy as jnp
from jax import lax
from jax.experimental import pallas as pl
from jax.experimental.pallas import tpu as pltpu
from jax.experimental.pallas import tpu_sc as plsc

N = 10000
D = 128
NP = 10112
ROWS_PER_TILE = NP // 16
E = 320000
CHUNK = 128
CHUNKS_PER_TILE = 80
EDGES_PER_TILE = CHUNKS_PER_TILE * CHUNK
EP = EDGES_PER_TILE * 32
ROW_BLOCK = 632
CHUNK2 = 128
TILE_ROWS2 = EDGES_PER_TILE // CHUNK2
PHASE_ROWS = 40
NPHASE = TILE_ROWS2 // PHASE_ROWS


def _linear_body(x_ref, w_ref, b_ref, c_ref, o_ref):
    acc = lax.dot_general(x_ref[...], w_ref[...], (((1,), (1,)), ((), ())),
                          preferred_element_type=jnp.float32)
    o_ref[...] = (acc + b_ref[...]) * c_ref[0]


def _linear(x, W, b, c):
    return pl.pallas_call(
        _linear_body,
        grid=(NP // ROW_BLOCK,),
        in_specs=[
            pl.BlockSpec((ROW_BLOCK, D), lambda i: (i, 0)),
            pl.BlockSpec((D, D), lambda i: (0, 0)),
            pl.BlockSpec((1, D), lambda i: (0, 0)),
            pl.BlockSpec(memory_space=pltpu.SMEM),
        ],
        out_specs=pl.BlockSpec((ROW_BLOCK, D), lambda i: (i, 0)),
        out_shape=jax.ShapeDtypeStruct((NP, D), jnp.float32),
    )(x, W, b, c)


def _mid_body(p_ref, w_ref, b_ref, c_ref, o_ref):
    h = jnp.maximum(p_ref[0] + p_ref[1], 0.0)
    acc = lax.dot_general(h, w_ref[...], (((1,), (1,)), ((), ())),
                          preferred_element_type=jnp.float32)
    o_ref[...] = (acc + b_ref[...]) * c_ref[0]


def _mid(p, W, b, c):
    return pl.pallas_call(
        _mid_body,
        grid=(NP // ROW_BLOCK,),
        in_specs=[
            pl.BlockSpec((2, ROW_BLOCK, D), lambda i: (0, i, 0)),
            pl.BlockSpec((D, D), lambda i: (0, 0)),
            pl.BlockSpec((1, D), lambda i: (0, 0)),
            pl.BlockSpec(memory_space=pltpu.SMEM),
        ],
        out_specs=pl.BlockSpec((ROW_BLOCK, D), lambda i: (i, 0)),
        out_shape=jax.ShapeDtypeStruct((NP, D), jnp.float32),
    )(p, W, b, c)


def _final_body(q_ref, o_ref):
    o = q_ref[0] + q_ref[1]
    m = jnp.max(o, axis=1, keepdims=True)
    e = jnp.exp(o - m)
    s = jnp.sum(e, axis=1, keepdims=True)
    o_ref[...] = o - m - jnp.log(s)


def _final(q):
    return pl.pallas_call(
        _final_body,
        grid=(NP // ROW_BLOCK,),
        in_specs=[pl.BlockSpec((2, ROW_BLOCK, D), lambda i: (0, i, 0))],
        out_specs=pl.BlockSpec((ROW_BLOCK, D), lambda i: (i, 0)),
        out_shape=jax.ShapeDtypeStruct((NP, D), jnp.float32),
    )(q)


_SC_MESH = plsc.VectorSubcoreMesh(core_axis_name="c", subcore_axis_name="s")


@functools.partial(
    pl.kernel,
    mesh=_SC_MESH,
    out_type=jax.ShapeDtypeStruct((2, NP, D), jnp.float32),
    scratch_types=[
        pltpu.VMEM((PHASE_ROWS, CHUNK2), jnp.int32),
        pltpu.VMEM((PHASE_ROWS, CHUNK2), jnp.int32),
        pltpu.VMEM((2, CHUNK2, D), jnp.float32),
        pltpu.VMEM_SHARED((NP, D), jnp.float32),
        pltpu.SemaphoreType.DMA,
        pltpu.SemaphoreType.DMA((2,)),
    ],
)
def _sc_scatter(y_hbm, z_hbm, src2d_hbm, dst2d_hbm, out_hbm,
                sidx, didx, rows, accum, gsem, ssem):
    c = lax.axis_index("c")
    s = lax.axis_index("s")
    wid = s * 2 + c
    r0 = s * ROWS_PER_TILE
    ch0 = wid * TILE_ROWS2

    @pl.when(c == 0)
    def _():
        pltpu.sync_copy(y_hbm.at[pl.ds(r0, ROWS_PER_TILE)],
                        accum.at[pl.ds(r0, ROWS_PER_TILE)])

    @pl.when(c == 1)
    def _():
        pltpu.sync_copy(z_hbm.at[pl.ds(r0, ROWS_PER_TILE)],
                        accum.at[pl.ds(r0, ROWS_PER_TILE)])

    plsc.subcore_barrier()

    def fire_scatter(kk, j):
        pltpu.async_copy(rows.at[j], accum.at[didx.at[kk]], ssem.at[j],
                         add=True)

    def wait_scatter(kk, j):
        pltpu.make_async_copy(rows.at[j], accum.at[didx.at[kk]],
                              ssem.at[j]).wait()

    for phase in range(NPHASE):
        pltpu.sync_copy(
            src2d_hbm.at[pl.ds(ch0 + phase * PHASE_ROWS, PHASE_ROWS)], sidx)
        pltpu.sync_copy(
            dst2d_hbm.at[pl.ds(ch0 + phase * PHASE_ROWS, PHASE_ROWS)], didx)

        def step(i, carry):
            for j in range(2):
                kk = 2 * i + j

                @pl.when(i > 0)
                def _():
                    wait_scatter(kk - 2, j)

                pltpu.async_copy(y_hbm.at[sidx.at[kk]], rows.at[j],
                                 gsem).wait()
                fire_scatter(kk, j)
            return carry

        lax.fori_loop(0, PHASE_ROWS // 2, step, 0)
        wait_scatter(PHASE_ROWS - 2, 0)
        wait_scatter(PHASE_ROWS - 1, 1)

    plsc.subcore_barrier()
    pltpu.sync_copy(accum.at[pl.ds(r0, ROWS_PER_TILE)],
                    out_hbm.at[c].at[pl.ds(r0, ROWS_PER_TILE)])


def kernel(x, edge_index, W1, b1, Wa1, ba1, c1, W2, b2, Wa2, ba2, c2):
    src = edge_index[0]
    dst = edge_index[1]
    pad_e = EP - E
    src_p = jnp.concatenate([src, jnp.zeros((pad_e,), src.dtype)])
    dst_p = jnp.concatenate([dst, jnp.full((pad_e,), N, dst.dtype)])
    src2d = src_p.reshape(TILE_ROWS2 * 32, CHUNK2)
    dst2d = dst_p.reshape(TILE_ROWS2 * 32, CHUNK2)

    x_p = jnp.pad(x, ((0, NP - N), (0, 0)))
    z = jnp.zeros((NP, D), jnp.float32)
    c1s = -jnp.abs(c1)
    c2s = -jnp.abs(c2)

    y1 = _linear(x_p, W1, b1.reshape(1, D), c1s)
    p = _sc_scatter(y1, z, src2d, dst2d)
    y2 = _mid(p, W2, b2.reshape(1, D), c2s)
    q = _sc_scatter(y2, z, src2d, dst2d)
    o = _final(q)
    return o[:N]


# speedup vs baseline: 1.3207x; 1.0335x over previous
import functools

import jax
import jax.numpy as jnp
from jax import lax
from jax.experimental import pallas as pl
from jax.experimental.pallas import tpu as pltpu
from jax.experimental.pallas import tpu_sc as plsc

N = 10000
D = 128
NP = 10112
ROWS_PER_TILE = NP // 16
E = 320000
CHUNK = 128
CHUNKS_PER_TILE = 80
EDGES_PER_TILE = CHUNKS_PER_TILE * CHUNK
EP = EDGES_PER_TILE * 32
ROW_BLOCK = 632
CHUNK2 = 128
TILE_ROWS2 = EDGES_PER_TILE // CHUNK2
PHASE_ROWS = 40
NPHASE = TILE_ROWS2 // PHASE_ROWS


def _linear_body(x_ref, w_ref, b_ref, c_ref, o_ref):
    acc = lax.dot_general(x_ref[...], w_ref[...], (((1,), (1,)), ((), ())),
                          preferred_element_type=jnp.float32)
    o_ref[...] = (acc + b_ref[...]) * c_ref[0]


def _linear(x, W, b, c):
    return pl.pallas_call(
        _linear_body,
        grid=(NP // ROW_BLOCK,),
        in_specs=[
            pl.BlockSpec((ROW_BLOCK, D), lambda i: (i, 0)),
            pl.BlockSpec((D, D), lambda i: (0, 0)),
            pl.BlockSpec((1, D), lambda i: (0, 0)),
            pl.BlockSpec(memory_space=pltpu.SMEM),
        ],
        out_specs=pl.BlockSpec((ROW_BLOCK, D), lambda i: (i, 0)),
        out_shape=jax.ShapeDtypeStruct((NP, D), jnp.float32),
    )(x, W, b, c)


def _mid_body(p_ref, w_ref, b_ref, c_ref, o_ref):
    h = jnp.maximum(p_ref[0] + p_ref[1], 0.0)
    acc = lax.dot_general(h, w_ref[...], (((1,), (1,)), ((), ())),
                          preferred_element_type=jnp.float32)
    o_ref[...] = (acc + b_ref[...]) * c_ref[0]


def _mid(p, W, b, c):
    return pl.pallas_call(
        _mid_body,
        grid=(NP // ROW_BLOCK,),
        in_specs=[
            pl.BlockSpec((2, ROW_BLOCK, D), lambda i: (0, i, 0)),
            pl.BlockSpec((D, D), lambda i: (0, 0)),
            pl.BlockSpec((1, D), lambda i: (0, 0)),
            pl.BlockSpec(memory_space=pltpu.SMEM),
        ],
        out_specs=pl.BlockSpec((ROW_BLOCK, D), lambda i: (i, 0)),
        out_shape=jax.ShapeDtypeStruct((NP, D), jnp.float32),
    )(p, W, b, c)


def _final_body(q_ref, o_ref):
    o = q_ref[0] + q_ref[1]
    m = jnp.max(o, axis=1, keepdims=True)
    e = jnp.exp(o - m)
    s = jnp.sum(e, axis=1, keepdims=True)
    o_ref[...] = o - m - jnp.log(s)


def _final(q):
    return pl.pallas_call(
        _final_body,
        grid=(NP // ROW_BLOCK,),
        in_specs=[pl.BlockSpec((2, ROW_BLOCK, D), lambda i: (0, i, 0))],
        out_specs=pl.BlockSpec((ROW_BLOCK, D), lambda i: (i, 0)),
        out_shape=jax.ShapeDtypeStruct((NP, D), jnp.float32),
    )(q)


_SC_MESH = plsc.VectorSubcoreMesh(core_axis_name="c", subcore_axis_name="s")


@functools.partial(
    pl.kernel,
    mesh=_SC_MESH,
    out_type=jax.ShapeDtypeStruct((2, NP, D), jnp.float32),
    scratch_types=[
        pltpu.VMEM((PHASE_ROWS, CHUNK2), jnp.int32),
        pltpu.VMEM((PHASE_ROWS, CHUNK2), jnp.int32),
        pltpu.VMEM((2, CHUNK2, D), jnp.float32),
        pltpu.VMEM_SHARED((NP, D), jnp.float32),
        pltpu.SemaphoreType.DMA((2,)),
    ],
)
def _sc_scatter(y_hbm, z_hbm, src2d_hbm, dst2d_hbm, out_hbm,
                sidx, didx, rows, accum, gsem):
    c = lax.axis_index("c")
    s = lax.axis_index("s")
    wid = s * 2 + c
    r0 = s * ROWS_PER_TILE
    ch0 = wid * TILE_ROWS2

    @pl.when(c == 0)
    def _():
        pltpu.sync_copy(y_hbm.at[pl.ds(r0, ROWS_PER_TILE)],
                        accum.at[pl.ds(r0, ROWS_PER_TILE)])

    @pl.when(c == 1)
    def _():
        pltpu.sync_copy(z_hbm.at[pl.ds(r0, ROWS_PER_TILE)],
                        accum.at[pl.ds(r0, ROWS_PER_TILE)])

    plsc.subcore_barrier()

    def fire_gather(kk, j):
        pltpu.async_copy(y_hbm.at[sidx.at[kk]], rows.at[j], gsem.at[j])

    def wait_gather(kk, j):
        pltpu.make_async_copy(y_hbm.at[sidx.at[kk]], rows.at[j],
                              gsem.at[j]).wait()

    for phase in range(NPHASE):
        pltpu.sync_copy(
            src2d_hbm.at[pl.ds(ch0 + phase * PHASE_ROWS, PHASE_ROWS)], sidx)
        pltpu.sync_copy(
            dst2d_hbm.at[pl.ds(ch0 + phase * PHASE_ROWS, PHASE_ROWS)], didx)

        fire_gather(0, 0)
        fire_gather(1, 1)

        def step(i, carry):
            for j in range(2):
                kk = 2 * i + j
                wait_gather(kk, j)
                pltpu.sync_copy(rows.at[j], accum.at[didx.at[kk]], add=True)

                @pl.when(kk + 2 < PHASE_ROWS)
                def _():
                    fire_gather(kk + 2, j)
            return carry

        lax.fori_loop(0, PHASE_ROWS // 2, step, 0)

    plsc.subcore_barrier()
    pltpu.sync_copy(accum.at[pl.ds(r0, ROWS_PER_TILE)],
                    out_hbm.at[c].at[pl.ds(r0, ROWS_PER_TILE)])


def kernel(x, edge_index, W1, b1, Wa1, ba1, c1, W2, b2, Wa2, ba2, c2):
    src = edge_index[0]
    dst = edge_index[1]
    pad_e = EP - E
    src_p = jnp.concatenate([src, jnp.zeros((pad_e,), src.dtype)])
    dst_p = jnp.concatenate([dst, jnp.full((pad_e,), N, dst.dtype)])
    src2d = src_p.reshape(TILE_ROWS2 * 32, CHUNK2)
    dst2d = dst_p.reshape(TILE_ROWS2 * 32, CHUNK2)

    x_p = jnp.pad(x, ((0, NP - N), (0, 0)))
    z = jnp.zeros((NP, D), jnp.float32)
    c1s = -jnp.abs(c1)
    c2s = -jnp.abs(c2)

    y1 = _linear(x_p, W1, b1.reshape(1, D), c1s)
    p = _sc_scatter(y1, z, src2d, dst2d)
    y2 = _mid(p, W2, b2.reshape(1, D), c2s)
    q = _sc_scatter(y2, z, src2d, dst2d)
    o = _final(q)
    return o[:N]


# speedup vs baseline: 1.3433x; 1.0171x over previous
import functools

import jax
import jax.numpy as jnp
from jax import lax
from jax.experimental import pallas as pl
from jax.experimental.pallas import tpu as pltpu
from jax.experimental.pallas import tpu_sc as plsc

N = 10000
D = 128
NP = 10240
ROWS_PER_TILE = NP // 16
E = 320000
CHUNK = 128
CHUNKS_PER_TILE = 80
EDGES_PER_TILE = CHUNKS_PER_TILE * CHUNK
EP = EDGES_PER_TILE * 32
ROW_BLOCK = 640


def _linear_body(x_ref, w_ref, b_ref, c_ref, o_ref):
    acc = lax.dot_general(x_ref[...], w_ref[...], (((1,), (1,)), ((), ())),
                          preferred_element_type=jnp.float32)
    o_ref[...] = (acc + b_ref[...]) * c_ref[0]


def _linear(x, W, b, c):
    return pl.pallas_call(
        _linear_body,
        grid=(NP // ROW_BLOCK,),
        in_specs=[
            pl.BlockSpec((ROW_BLOCK, D), lambda i: (i, 0)),
            pl.BlockSpec((D, D), lambda i: (0, 0)),
            pl.BlockSpec((1, D), lambda i: (0, 0)),
            pl.BlockSpec(memory_space=pltpu.SMEM),
        ],
        out_specs=pl.BlockSpec((ROW_BLOCK, D), lambda i: (i, 0)),
        out_shape=jax.ShapeDtypeStruct((NP, D), jnp.float32),
    )(x, W, b, c)


def _mid_body(p_ref, w_ref, b_ref, c_ref, o_ref):
    h = jnp.maximum(p_ref[0] + p_ref[1], 0.0)
    acc = lax.dot_general(h, w_ref[...], (((1,), (1,)), ((), ())),
                          preferred_element_type=jnp.float32)
    o_ref[...] = (acc + b_ref[...]) * c_ref[0]


def _mid(p, W, b, c):
    return pl.pallas_call(
        _mid_body,
        grid=(NP // ROW_BLOCK,),
        in_specs=[
            pl.BlockSpec((2, ROW_BLOCK, D), lambda i: (0, i, 0)),
            pl.BlockSpec((D, D), lambda i: (0, 0)),
            pl.BlockSpec((1, D), lambda i: (0, 0)),
            pl.BlockSpec(memory_space=pltpu.SMEM),
        ],
        out_specs=pl.BlockSpec((ROW_BLOCK, D), lambda i: (i, 0)),
        out_shape=jax.ShapeDtypeStruct((NP, D), jnp.float32),
    )(p, W, b, c)


def _final_body(q_ref, o_ref):
    o = q_ref[0] + q_ref[1]
    m = jnp.max(o, axis=1, keepdims=True)
    e = jnp.exp(o - m)
    s = jnp.sum(e, axis=1, keepdims=True)
    o_ref[...] = o - m - jnp.log(s)


def _final(q):
    return pl.pallas_call(
        _final_body,
        grid=(N // 2000,),
        in_specs=[pl.BlockSpec((2, 2000, D), lambda i: (0, i, 0))],
        out_specs=pl.BlockSpec((2000, D), lambda i: (i, 0)),
        out_shape=jax.ShapeDtypeStruct((N, D), jnp.float32),
    )(q)


_SC_MESH = plsc.VectorSubcoreMesh(core_axis_name="c", subcore_axis_name="s")


@functools.partial(
    pl.kernel,
    mesh=_SC_MESH,
    out_type=jax.ShapeDtypeStruct((2, NP, D), jnp.float32),
    scratch_types=[
        pltpu.VMEM((EDGES_PER_TILE,), jnp.int32),
        pltpu.VMEM((CHUNKS_PER_TILE, CHUNK), jnp.int32),
        pltpu.VMEM((CHUNK, D), jnp.float32),
        pltpu.VMEM_SHARED((NP, D), jnp.float32),
        pltpu.SemaphoreType.DMA,
    ],
)
def _sc_scatter(y_hbm, z_hbm, src_hbm, dst2d_hbm, out_hbm,
                sidx, didx, rows, accum, sem):
    c = lax.axis_index("c")
    s = lax.axis_index("s")
    wid = s * 2 + c
    r0 = s * ROWS_PER_TILE

    @pl.when(c == 0)
    def _():
        pltpu.sync_copy(y_hbm.at[pl.ds(r0, ROWS_PER_TILE)],
                        accum.at[pl.ds(r0, ROWS_PER_TILE)])

    @pl.when(c == 1)
    def _():
        pltpu.sync_copy(z_hbm.at[pl.ds(r0, ROWS_PER_TILE)],
                        accum.at[pl.ds(r0, ROWS_PER_TILE)])

    plsc.subcore_barrier()

    pltpu.sync_copy(src_hbm.at[pl.ds(wid * EDGES_PER_TILE, EDGES_PER_TILE)],
                    sidx)
    pltpu.sync_copy(dst2d_hbm.at[pl.ds(wid * CHUNKS_PER_TILE, CHUNKS_PER_TILE)],
                    didx)

    def step(k, carry):
        pltpu.async_copy(y_hbm.at[sidx.at[pl.ds(k * CHUNK, CHUNK)]],
                         rows, sem).wait()
        pltpu.sync_copy(rows, accum.at[didx.at[k]], add=True)
        return carry

    lax.fori_loop(0, CHUNKS_PER_TILE, step, 0)

    plsc.subcore_barrier()
    pltpu.sync_copy(accum.at[pl.ds(r0, ROWS_PER_TILE)],
                    out_hbm.at[c].at[pl.ds(r0, ROWS_PER_TILE)])


def kernel(x, edge_index, W1, b1, Wa1, ba1, c1, W2, b2, Wa2, ba2, c2):
    src = edge_index[0]
    dst = edge_index[1]
    pad_e = EP - E
    src_p = jnp.concatenate([src, jnp.zeros((pad_e,), src.dtype)])
    dst_p = jnp.concatenate([dst, jnp.full((pad_e,), N, dst.dtype)])
    dst2d = dst_p.reshape(CHUNKS_PER_TILE * 32, CHUNK)

    x_p = jnp.pad(x, ((0, NP - N), (0, 0)))
    z = jnp.zeros((NP, D), jnp.float32)
    c1s = -jnp.abs(c1)
    c2s = -jnp.abs(c2)

    y1 = _linear(x_p, W1, b1.reshape(1, D), c1s)
    p = _sc_scatter(y1, z, src_p, dst2d)
    y2 = _mid(p, W2, b2.reshape(1, D), c2s)
    q = _sc_scatter(y2, z, src_p, dst2d)
    return _final(q)
